# baseline (speedup 1.0000x reference)
import functools
import numpy as np
import jax
import jax.numpy as jnp
from jax import lax
from jax.experimental import pallas as pl
from jax.experimental.pallas import tpu as pltpu
from jax.experimental.pallas import tpu_sc as plsc

_NUM_CLASSES = 19
_CP = 24
_D = 256
_N = 1024
_BINS = 51
_BINS_VALS = np.linspace(-5.0, 5.0, _BINS).astype(np.float32)
_TWO_PI = 6.283185307179586
_LANES = 16
_WORKERS = 32
_PT = _N // _WORKERS


def _sc_body(featT_hbm, lbl_hbm, zrow_hbm, zcnt_hbm,
             part1_hbm, part2_hbm, partc_hbm,
             lbl_v, rows_v, acc1, acc2, accc, sem):
    cid = jnp.int32(lax.axis_index("c"))
    sid = jnp.int32(lax.axis_index("s"))
    wid = cid * (_WORKERS // 2) + sid

    base = wid * _PT
    pltpu.sync_copy(lbl_hbm.at[pl.ds(base, _PT)], lbl_v)
    pltpu.sync_copy(featT_hbm.at[pl.ds(base, _PT)], rows_v)
    pltpu.sync_copy(zrow_hbm, acc1)
    pltpu.sync_copy(zrow_hbm, acc2)
    pltpu.sync_copy(zcnt_hbm, accc)

    lane = lax.broadcasted_iota(jnp.int32, (_LANES,), 0)
    onev = jnp.full((_LANES,), 1.0, jnp.float32)

    def _pix(i, carry):
        cls = plsc.load_gather(lbl_v, [jnp.full((_LANES,), i, jnp.int32)])
        plsc.addupdate_scatter(accc, [cls, lane], onev)
        for j in range(_D // _LANES):
            x = rows_v[i, pl.ds(j * _LANES, _LANES)]
            col = lane + jnp.int32(j * _LANES)
            plsc.addupdate_scatter(acc1, [cls, col], x)
            plsc.addupdate_scatter(acc2, [cls, col], x * x)
        return carry

    lax.fori_loop(jnp.int32(0), jnp.int32(_PT), _pix, jnp.int32(0))

    pltpu.sync_copy(acc1, part1_hbm.at[pl.ds(wid * _CP, _CP)])
    pltpu.sync_copy(acc2, part2_hbm.at[pl.ds(wid * _CP, _CP)])
    pltpu.sync_copy(accc, partc_hbm.at[pl.ds(wid * _CP, _CP)])


_sc_kernel = functools.partial(
    pl.kernel,
    mesh=plsc.VectorSubcoreMesh(core_axis_name="c", subcore_axis_name="s"),
    compiler_params=pltpu.CompilerParams(needs_layout_passes=False),
    out_type=[
        jax.ShapeDtypeStruct((_WORKERS * _CP, _D), jnp.float32),
        jax.ShapeDtypeStruct((_WORKERS * _CP, _D), jnp.float32),
        jax.ShapeDtypeStruct((_WORKERS * _CP, _LANES), jnp.float32),
    ],
    scratch_types=[
        pltpu.VMEM((_PT,), jnp.int32),
        pltpu.VMEM((_PT, _D), jnp.float32),
        pltpu.VMEM((_CP, _D), jnp.float32),
        pltpu.VMEM((_CP, _D), jnp.float32),
        pltpu.VMEM((_CP, _LANES), jnp.float32),
        pltpu.SemaphoreType.DMA,
    ],
)(_sc_body)


def _dot(a, b, precision):
    return lax.dot_general(
        a, b, (((1,), (0,)), ((), ())),
        precision=precision, preferred_element_type=jnp.float32)


def _tc_body(featT_ref, lblrow_ref, lblcol_ref, bins_ref,
             part1_ref, part2_ref, partc_ref, out_ref,
             sample_scr, target_scr):
    featT = featT_ref[...]
    lbl_row = lblrow_ref[...]
    lbl_col = lblcol_ref[...]

    s1 = jnp.sum(part1_ref[...], axis=0)[:_NUM_CLASSES]
    s2 = jnp.sum(part2_ref[...], axis=0)[:_NUM_CLASSES]
    cnt = jnp.sum(partc_ref[...], axis=0)[:_NUM_CLASSES, :1]

    onehot = (lax.broadcasted_iota(jnp.int32, (_NUM_CLASSES, _N), 0)
              == lbl_row).astype(jnp.float32)
    onehotT = (lax.broadcasted_iota(jnp.int32, (_N, _NUM_CLASSES), 1)
               == lbl_col).astype(jnp.float32)

    inv_n = 1.0 / jnp.maximum(cnt, 1.0)
    miu = s1 * inv_n
    var = jnp.maximum(s2 * inv_n - miu * miu, 1e-12)
    vs = var * (1.0 / 25.0)
    coef_s = -0.5 / vs
    nrm_s = lax.rsqrt(_TWO_PI * vs)
    coef_t = -0.5 / var
    nrm_t = lax.rsqrt(_TWO_PI * var)

    hi = lax.Precision.HIGHEST
    coef_pix = _dot(onehotT, coef_s, hi)
    nrm_pix = _dot(onehotT, nrm_s, hi)

    onehot_bf = onehot.astype(jnp.bfloat16)

    def bin_group(g, carry):
        for u in range(3):
            b = g * jnp.int32(3) + jnp.int32(u)
            bv = bins_ref[b]
            d = bv - featT
            kern = jnp.exp(d * d * coef_pix) * nrm_pix
            sam = _dot(onehot_bf, kern.astype(jnp.bfloat16),
                       lax.Precision.DEFAULT)
            sample_scr[pl.ds(b, 1)] = sam[None]
            dt = bv - miu
            tgt = jnp.exp(dt * dt * coef_t) * nrm_t
            target_scr[pl.ds(b, 1)] = tgt[None]
        return carry

    lax.fori_loop(jnp.int32(0), jnp.int32(_BINS // 3), bin_group, 0)

    sample = sample_scr[...]
    target = target_scr[...]
    inv_zs = 1.0 / jnp.maximum(jnp.sum(sample, axis=0), 1e-20)
    inv_zt = 1.0 / jnp.maximum(jnp.sum(target, axis=0), 1e-20)
    dd = sample * inv_zs[None] - target * inv_zt[None]
    sl1 = jnp.where(jnp.abs(dd) < 1.0, 0.5 * dd * dd, jnp.abs(dd) - 0.5)
    per_c = jnp.sum(jnp.sum(sl1, axis=0), axis=1, keepdims=True)
    cls = lax.broadcasted_iota(jnp.int32, (_NUM_CLASSES, 1), 0)
    gated = jnp.where((cnt > 0.0) & (cls > 0), per_c, 0.0)
    out_ref[0, 0] = jnp.sum(gated) * (1.0 / (_D * _BINS))


def kernel(feature, label):
    B, D, H, W = feature.shape
    sh = label.shape[2] // H
    sw = label.shape[3] // W
    lbl = label[0, 0, ::sh, ::sw].reshape(-1).astype(jnp.int32)
    featT = feature[0].reshape(D, -1).T
    n = featT.shape[0]

    zrow = jnp.zeros((_CP, _D), jnp.float32)
    zcnt = jnp.zeros((_CP, _LANES), jnp.float32)
    part1, part2, partc = _sc_kernel(featT, lbl, zrow, zcnt)

    out = pl.pallas_call(
        _tc_body,
        out_shape=jax.ShapeDtypeStruct((1, 1), jnp.float32),
        out_specs=pl.BlockSpec(memory_space=pltpu.MemorySpace.SMEM),
        in_specs=[
            pl.BlockSpec(memory_space=pltpu.MemorySpace.VMEM),
            pl.BlockSpec(memory_space=pltpu.MemorySpace.VMEM),
            pl.BlockSpec(memory_space=pltpu.MemorySpace.VMEM),
            pl.BlockSpec(memory_space=pltpu.MemorySpace.SMEM),
            pl.BlockSpec(memory_space=pltpu.MemorySpace.VMEM),
            pl.BlockSpec(memory_space=pltpu.MemorySpace.VMEM),
            pl.BlockSpec(memory_space=pltpu.MemorySpace.VMEM),
        ],
        scratch_shapes=[
            pltpu.VMEM((_BINS, _NUM_CLASSES, _D), jnp.float32),
            pltpu.VMEM((_BINS, _NUM_CLASSES, _D), jnp.float32),
        ],
    )(featT, lbl.reshape(1, n), lbl.reshape(n, 1), jnp.asarray(_BINS_VALS),
      part1.reshape(_WORKERS, _CP, _D), part2.reshape(_WORKERS, _CP, _D),
      partc.reshape(_WORKERS, _CP, _LANES))
    return out[0, 0]
